# baseline (speedup 1.0000x reference)
import functools

import jax
import jax.numpy as jnp
from jax import lax
from jax.experimental import pallas as pl
from jax.experimental.pallas import tpu as pltpu
from jax.experimental.pallas import tpu_sc as plsc

VOCAB = 1000000
D = 64
OUT_DIM = 1000
B = 4096

NUM_WORKERS = 32
ROWS_PER_W = B // NUM_WORKERS
NAME_L = 20
DESC_L = 200
TOT_L = NAME_L + DESC_L
CHUNK = TOT_L // 2
NBUF = 4


def _sc_embed_sums(cat_idxs, emb_table):
    mesh = plsc.VectorSubcoreMesh(core_axis_name="c", subcore_axis_name="s")

    @functools.partial(
        pl.kernel,
        out_type=jax.ShapeDtypeStruct((B, 2 * D), jnp.float32),
        mesh=mesh,
        compiler_params=pltpu.CompilerParams(
            use_tc_tiling_on_sc=False, needs_layout_passes=False),
        scratch_types=[
            pltpu.VMEM((ROWS_PER_W, 2, CHUNK), jnp.int32),
            pltpu.VMEM((NBUF, TOT_L, D), jnp.float32),
            pltpu.VMEM((ROWS_PER_W, 2 * D), jnp.float32),
            pltpu.SemaphoreType.DMA((NBUF,)),
        ],
    )
    def body(idx_hbm, table_hbm, out_hbm, idx_v, bufs, outv, sems):
        wid = lax.axis_index("s") * 2 + lax.axis_index("c")
        base = wid * ROWS_PER_W
        pltpu.sync_copy(idx_hbm.at[pl.ds(base, ROWS_PER_W)], idx_v)

        def issue(g, slot):
            pltpu.async_copy(table_hbm.at[idx_v.at[g, 0]],
                             bufs.at[slot, pl.ds(0, CHUNK)], sems.at[slot])
            pltpu.async_copy(table_hbm.at[idx_v.at[g, 1]],
                             bufs.at[slot, pl.ds(CHUNK, CHUNK)], sems.at[slot])

        for p in range(NBUF - 1):
            issue(p, p)

        def process_row(r):
            g = r + NBUF - 1

            @pl.when(g < ROWS_PER_W)
            def _():
                issue(g, lax.rem(g, NBUF))

            slot = lax.rem(r, NBUF)
            pltpu.make_async_copy(table_hbm.at[pl.ds(0, TOT_L)],
                                  bufs.at[slot], sems.at[slot]).wait()
            for c in range(D // 16):
                sl = pl.ds(c * 16, 16)
                acc_n = bufs[slot, 0, sl]
                for j in range(1, NAME_L):
                    acc_n = acc_n + bufs[slot, j, sl]
                outv[r, pl.ds(c * 16, 16)] = acc_n
                acc_d = bufs[slot, NAME_L, sl]
                for j in range(NAME_L + 1, TOT_L):
                    acc_d = acc_d + bufs[slot, j, sl]
                outv[r, pl.ds(D + c * 16, 16)] = acc_d

        pl.loop(0, ROWS_PER_W)(process_row)
        pltpu.sync_copy(outv, out_hbm.at[pl.ds(base, ROWS_PER_W)])

    return body(cat_idxs, emb_table)


def _tc_norm_fc(sums, fc_w, fc_b):
    BT = 512

    def body(s_ref, w_ref, b_ref, o_ref):
        s = s_ref[...]
        n = s[:, :D]
        d = s[:, D:]
        nss = jnp.sum(n * n, axis=1, keepdims=True)
        dss = jnp.sum(d * d, axis=1, keepdims=True)
        nn = n * lax.rsqrt(jnp.maximum(nss, 1e-24))
        dn = d * lax.rsqrt(jnp.maximum(dss, 1e-24))
        x = jnp.concatenate([nn, dn], axis=1)
        o_ref[...] = (
            lax.dot_general(x, w_ref[...], (((1,), (1,)), ((), ())),
                            preferred_element_type=jnp.float32)
            + b_ref[...]
        )

    return pl.pallas_call(
        body,
        grid=(B // BT,),
        in_specs=[
            pl.BlockSpec((BT, 2 * D), lambda i: (i, 0)),
            pl.BlockSpec((OUT_DIM, 2 * D), lambda i: (0, 0)),
            pl.BlockSpec((1, OUT_DIM), lambda i: (0, 0)),
        ],
        out_specs=pl.BlockSpec((BT, OUT_DIM), lambda i: (i, 0)),
        out_shape=jax.ShapeDtypeStruct((B, OUT_DIM), jnp.float32),
    )(sums, fc_w, fc_b.reshape(1, OUT_DIM))


def kernel(name_idxs, name_len, desc_idxs, desc_len, union_idxs, union_len, emb_table, fc_w, fc_b):
    cat_idxs = jnp.concatenate([name_idxs, desc_idxs], axis=1).reshape(B, 2, CHUNK)
    sums = _sc_embed_sums(cat_idxs, emb_table)
    return _tc_norm_fc(sums, fc_w, fc_b)


# speedup vs baseline: 1.0396x; 1.0396x over previous
import functools

import jax
import jax.numpy as jnp
import numpy as np
from jax import lax
from jax.experimental import pallas as pl
from jax.experimental.pallas import tpu as pltpu
from jax.experimental.pallas import tpu_sc as plsc

VOCAB = 1000000
D = 64
OUT_DIM = 1000
B = 4096

NUM_WORKERS = 32
ROWS_PER_W = B // NUM_WORKERS
NAME_L = 20
DESC_L = 200
TOT_L = NAME_L + DESC_L
CHUNK = TOT_L // 2
NBUF = 6

_PERM = np.concatenate(
    [np.concatenate([g * 32 + 2 * np.arange(16), g * 32 + 2 * np.arange(16) + 1])
     for g in range(2)]
)


def _sc_embed_sums(cat_idxs, emb_table_bf16):
    mesh = plsc.VectorSubcoreMesh(core_axis_name="c", subcore_axis_name="s")

    @functools.partial(
        pl.kernel,
        out_type=jax.ShapeDtypeStruct((B, 2 * D), jnp.float32),
        mesh=mesh,
        compiler_params=pltpu.CompilerParams(
            use_tc_tiling_on_sc=False, needs_layout_passes=False),
        scratch_types=[
            pltpu.VMEM((ROWS_PER_W, 2, CHUNK), jnp.int32),
            pltpu.VMEM((NBUF, TOT_L, D), jnp.bfloat16),
            pltpu.VMEM((ROWS_PER_W, 2 * D), jnp.float32),
            pltpu.SemaphoreType.DMA((NBUF,)),
        ],
    )
    def body(idx_hbm, table_hbm, out_hbm, idx_v, bufs, outv, sems):
        wid = lax.axis_index("s") * 2 + lax.axis_index("c")
        base = wid * ROWS_PER_W
        pltpu.sync_copy(idx_hbm.at[pl.ds(base, ROWS_PER_W)], idx_v)

        def issue(g, slot):
            pltpu.async_copy(table_hbm.at[idx_v.at[g, 0]],
                             bufs.at[slot, pl.ds(0, CHUNK)], sems.at[slot])
            pltpu.async_copy(table_hbm.at[idx_v.at[g, 1]],
                             bufs.at[slot, pl.ds(CHUNK, CHUNK)], sems.at[slot])

        for p in range(NBUF - 1):
            issue(p, p)

        def process_row(r):
            g = r + NBUF - 1

            @pl.when(g < ROWS_PER_W)
            def _():
                issue(g, lax.rem(g, NBUF))

            slot = lax.rem(r, NBUF)
            pltpu.make_async_copy(table_hbm.at[pl.ds(0, TOT_L)],
                                  bufs.at[slot], sems.at[slot]).wait()

            zero = jnp.zeros((16,), jnp.float32)
            acc = [[[zero, zero] for _ in range(2)] for _ in range(2)]
            for j in range(NAME_L):
                for c2 in range(2):
                    a, b_ = plsc.unpack(bufs[slot, j, pl.ds(c2 * 32, 32)],
                                        format=plsc.PackFormat.INTERLEAVED)
                    acc[0][c2][0] = acc[0][c2][0] + a
                    acc[0][c2][1] = acc[0][c2][1] + b_
            for j in range(NAME_L, TOT_L):
                for c2 in range(2):
                    a, b_ = plsc.unpack(bufs[slot, j, pl.ds(c2 * 32, 32)],
                                        format=plsc.PackFormat.INTERLEAVED)
                    acc[1][c2][0] = acc[1][c2][0] + a
                    acc[1][c2][1] = acc[1][c2][1] + b_
            for h in range(2):
                for c2 in range(2):
                    for eo in range(2):
                        outv[r, pl.ds(h * 64 + c2 * 32 + eo * 16, 16)] = acc[h][c2][eo]

        pl.loop(0, ROWS_PER_W)(process_row)
        pltpu.sync_copy(outv, out_hbm.at[pl.ds(base, ROWS_PER_W)])

    return body(cat_idxs, emb_table_bf16)


def _tc_norm_fc(sums, fc_w_perm, fc_b):
    BT = 512

    def body(s_ref, w_ref, b_ref, o_ref):
        s = s_ref[...]
        n = s[:, :D]
        d = s[:, D:]
        nss = jnp.sum(n * n, axis=1, keepdims=True)
        dss = jnp.sum(d * d, axis=1, keepdims=True)
        nn = n * lax.rsqrt(jnp.maximum(nss, 1e-24))
        dn = d * lax.rsqrt(jnp.maximum(dss, 1e-24))
        x = jnp.concatenate([nn, dn], axis=1)
        o_ref[...] = (
            lax.dot_general(x, w_ref[...], (((1,), (1,)), ((), ())),
                            preferred_element_type=jnp.float32)
            + b_ref[...]
        )

    return pl.pallas_call(
        body,
        grid=(B // BT,),
        in_specs=[
            pl.BlockSpec((BT, 2 * D), lambda i: (i, 0)),
            pl.BlockSpec((OUT_DIM, 2 * D), lambda i: (0, 0)),
            pl.BlockSpec((1, OUT_DIM), lambda i: (0, 0)),
        ],
        out_specs=pl.BlockSpec((BT, OUT_DIM), lambda i: (i, 0)),
        out_shape=jax.ShapeDtypeStruct((B, OUT_DIM), jnp.float32),
    )(sums, fc_w_perm, fc_b.reshape(1, OUT_DIM))


def kernel(name_idxs, name_len, desc_idxs, desc_len, union_idxs, union_len, emb_table, fc_w, fc_b):
    cat_idxs = jnp.concatenate([name_idxs, desc_idxs], axis=1).reshape(B, 2, CHUNK)
    emb_bf16 = emb_table.astype(jnp.bfloat16)
    sums = _sc_embed_sums(cat_idxs, emb_bf16)
    perm = jnp.asarray(np.concatenate([_PERM, _PERM + 64]), dtype=jnp.int32)
    fc_w_perm = fc_w[:, perm]
    return _tc_norm_fc(sums, fc_w_perm, fc_b)
